# baseline (speedup 1.0000x reference)
import functools

import jax
import jax.numpy as jnp
from jax import lax
from jax.experimental import pallas as pl
from jax.experimental.pallas import tpu as pltpu
from jax.experimental.pallas import tpu_sc as plsc

N_NODES = 10000
N_EDGES = 320000
DIM = 128
ALPHA = 0.2

NC = 2
NS = 16
NW = NC * NS
EPW = N_EDGES // NW
NPAD = 10240
ROWS_PER_SUB = NPAD // NS
B = 80
NB = EPW // B
NEG = -3.0e38

_mesh = plsc.VectorSubcoreMesh(core_axis_name="c", subcore_axis_name="s")
_sc_params = pltpu.CompilerParams(needs_layout_passes=False)


def _mm_body(x_ref, w_ref, o_ref):
    o_ref[...] = jnp.dot(x_ref[...], w_ref[...],
                         preferred_element_type=jnp.float32)


def _matmul(x, w_cat):
    return pl.pallas_call(
        _mm_body,
        out_shape=jax.ShapeDtypeStruct((N_NODES, 2 * DIM), jnp.float32),
        grid=(10,),
        in_specs=[
            pl.BlockSpec((N_NODES // 10, DIM), lambda i: (i, 0)),
            pl.BlockSpec((DIM, 2 * DIM), lambda i: (0, 0)),
        ],
        out_specs=pl.BlockSpec((N_NODES // 10, 2 * DIM), lambda i: (i, 0)),
    )(x, w_cat)


def _seg_fold(tidx, vals, acc_ref, iota, is_sum):
    sk, sv = plsc.sort_key_val(tidx, vals)
    for d in (1, 2, 4, 8):
        lane = jnp.maximum(iota - d, 0)
        kk = sk.at[lane].get(mode="promise_in_bounds")
        vv = sv.at[lane].get(mode="promise_in_bounds")
        take = (kk == sk) & (iota >= d)
        folded = sv + vv if is_sum else jnp.maximum(sv, vv)
        sv = jnp.where(take, folded, sv)
    nxt = sk.at[jnp.minimum(iota + 1, 15)].get(mode="promise_in_bounds")
    last = (sk != nxt) | (iota == 15)
    cur = plsc.load_gather(acc_ref, [sk], mask=last)
    upd = cur + sv if is_sum else jnp.maximum(cur, sv)
    plsc.store_scatter(acc_ref, [sk], upd, mask=last)


def _p2_body(s_hbm, t_hbm, src_hbm, tgt_hbm, e_hbm, m32_hbm,
             s_v, t_v, src_v, tgt_v, e_v, m_v):
    c = lax.axis_index("c")
    s = lax.axis_index("s")
    wid = s * NC + c
    base = wid * EPW
    pltpu.sync_copy(s_hbm, s_v)
    pltpu.sync_copy(t_hbm, t_v)
    pltpu.sync_copy(src_hbm.at[pl.ds(base, EPW)], src_v)
    pltpu.sync_copy(tgt_hbm.at[pl.ds(base, EPW)], tgt_v)

    def _init(i, _):
        m_v[pl.ds(i * 16, 16)] = jnp.full((16,), NEG, jnp.float32)
        return 0
    lax.fori_loop(0, NPAD // 16, _init, 0)

    iota = lax.iota(jnp.int32, 16)

    def _step(i, _):
        off = i * 16
        sidx = src_v[pl.ds(off, 16)]
        tidx = tgt_v[pl.ds(off, 16)]
        ev = plsc.load_gather(s_v, [sidx]) + plsc.load_gather(t_v, [tidx])
        ev = jnp.where(ev > 0, ev, ALPHA * ev)
        e_v[pl.ds(off, 16)] = ev
        _seg_fold(tidx, ev, m_v, iota, is_sum=False)
        return 0
    lax.fori_loop(0, EPW // 16, _step, 0)

    pltpu.sync_copy(e_v, e_hbm.at[pl.ds(base, EPW)])
    pltpu.sync_copy(m_v, m32_hbm.at[wid])


_p2 = pl.kernel(
    _p2_body,
    out_type=[jax.ShapeDtypeStruct((N_EDGES,), jnp.float32),
              jax.ShapeDtypeStruct((NW, NPAD), jnp.float32)],
    mesh=_mesh,
    compiler_params=_sc_params,
    scratch_types=[pltpu.VMEM((NPAD,), jnp.float32),
                   pltpu.VMEM((NPAD,), jnp.float32),
                   pltpu.VMEM((EPW,), jnp.int32),
                   pltpu.VMEM((EPW,), jnp.int32),
                   pltpu.VMEM((EPW,), jnp.float32),
                   pltpu.VMEM((NPAD,), jnp.float32)],
)


SLICE = NPAD // NS


def _p4_body(m32_hbm, e_hbm, tgt_hbm, ex_hbm, d32_hbm,
             mst, mred, m_v, e_v, tgt_v, ex_v, d_v, m_s):
    c = lax.axis_index("c")
    s = lax.axis_index("s")
    wid = s * NC + c
    base = wid * EPW
    pltpu.sync_copy(m32_hbm.at[:, pl.ds(s * SLICE, SLICE)], mst)
    pltpu.sync_copy(e_hbm.at[pl.ds(base, EPW)], e_v)
    pltpu.sync_copy(tgt_hbm.at[pl.ds(base, EPW)], tgt_v)

    def _red(i, _):
        off = i * 16
        acc = mst[0, pl.ds(off, 16)]
        for w in range(1, NW):
            acc = jnp.maximum(acc, mst[w, pl.ds(off, 16)])
        mred[pl.ds(off, 16)] = acc
        return 0
    lax.fori_loop(0, SLICE // 16, _red, 0)
    pltpu.sync_copy(mred, m_s.at[pl.ds(s * SLICE, SLICE)])

    def _init(i, _):
        d_v[pl.ds(i * 16, 16)] = jnp.zeros((16,), jnp.float32)
        return 0
    lax.fori_loop(0, NPAD // 16, _init, 0)

    plsc.subcore_barrier()
    pltpu.sync_copy(m_s, m_v)

    def _step(i, _):
        off = i * 16
        tidx = tgt_v[pl.ds(off, 16)]
        ev = e_v[pl.ds(off, 16)]
        ex = jnp.exp(ev - plsc.load_gather(m_v, [tidx]))
        ex_v[pl.ds(off, 16)] = ex
        plsc.addupdate_scatter(d_v, [tidx], ex)
        return 0
    lax.fori_loop(0, EPW // 16, _step, 0)

    pltpu.sync_copy(ex_v, ex_hbm.at[pl.ds(base, EPW)])
    pltpu.sync_copy(d_v, d32_hbm.at[wid])


_p4 = pl.kernel(
    _p4_body,
    out_type=[jax.ShapeDtypeStruct((N_EDGES,), jnp.float32),
              jax.ShapeDtypeStruct((NW, NPAD), jnp.float32)],
    mesh=_mesh,
    compiler_params=_sc_params,
    scratch_types=[pltpu.VMEM((NW, SLICE), jnp.float32),
                   pltpu.VMEM((SLICE,), jnp.float32),
                   pltpu.VMEM((NPAD,), jnp.float32),
                   pltpu.VMEM((EPW,), jnp.float32),
                   pltpu.VMEM((EPW,), jnp.int32),
                   pltpu.VMEM((EPW,), jnp.float32),
                   pltpu.VMEM((NPAD,), jnp.float32),
                   pltpu.VMEM_SHARED((NPAD,), jnp.float32)],
)


def _p5_body(h_hbm, ex_hbm, src_hbm, tgt_hbm, hp2_hbm,
             ex2, src_v, tgt_v, rows2, hp_s, gsem, esem, ssem):
    c = lax.axis_index("c")
    s = lax.axis_index("s")
    wid = s * NC + c
    base = wid * EPW
    pltpu.sync_copy(src_hbm.at[pl.ds(base, EPW)], src_v.at[pl.ds(0, EPW)])
    pltpu.sync_copy(tgt_hbm.at[pl.ds(base, EPW)], tgt_v)
    zer16 = jnp.zeros((16,), jnp.int32)
    for k in range(B // 16):
        src_v[pl.ds(EPW + k * 16, 16)] = zer16

    def _zinit(i, _):
        r = i // 8
        cc = i % 8
        rows2[0, r, pl.ds(cc * 16, 16)] = jnp.zeros((16,), jnp.float32)
        rows2[1, r, pl.ds(cc * 16, 16)] = jnp.zeros((16,), jnp.float32)
        return 0
    lax.fori_loop(0, B * 8, _zinit, 0)
    for k in range(ROWS_PER_SUB // B):
        pltpu.sync_copy(rows2.at[0],
                        hp_s.at[pl.ds(s * ROWS_PER_SUB + k * B, B)])
    plsc.subcore_barrier()

    pltpu.async_copy(rows2.at[1], hp_s.at[tgt_v.at[pl.ds(0, B)]], ssem,
                     add=True)
    pltpu.async_copy(h_hbm.at[src_v.at[pl.ds(0, B)]], rows2.at[0], gsem)
    pltpu.async_copy(ex_hbm.at[pl.ds(base, B)], ex2.at[pl.ds(0, B)], esem)

    def _process(i, p):
        off = i * B
        off_n = lax.rem(off + B, EPW)
        pltpu.make_async_copy(rows2.at[1 - p],
                              hp_s.at[tgt_v.at[pl.ds(off, B)]], ssem).wait()
        pltpu.async_copy(h_hbm.at[src_v.at[pl.ds(off + B, B)]],
                         rows2.at[1 - p], gsem)
        pltpu.async_copy(ex_hbm.at[pl.ds(base + off_n, B)],
                         ex2.at[pl.ds((1 - p) * B, B)], esem)
        pltpu.make_async_copy(h_hbm.at[src_v.at[pl.ds(off, B)]],
                              rows2.at[p], gsem).wait()
        pltpu.make_async_copy(ex_hbm.at[pl.ds(base + off, B)],
                              ex2.at[pl.ds(p * B, B)], esem).wait()

        for g in range(B // 16):
            ex16 = ex2[pl.ds(p * B + g * 16, 16)]
            for r16 in range(16):
                lane = jnp.full((16,), r16, jnp.int32)
                bc = ex16.at[lane].get(mode="promise_in_bounds")
                r = g * 16 + r16
                for cc in range(8):
                    rows2[p, r, pl.ds(cc * 16, 16)] = (
                        rows2[p, r, pl.ds(cc * 16, 16)] * bc)

        pltpu.async_copy(rows2.at[p], hp_s.at[tgt_v.at[pl.ds(off, B)]],
                        ssem, add=True)

    def _pair(j, _):
        _process(2 * j, 0)
        _process(2 * j + 1, 1)
        return 0
    lax.fori_loop(0, NB // 2, _pair, 0)
    _process(NB - 1, 0)

    pltpu.make_async_copy(h_hbm.at[src_v.at[pl.ds(NB * B, B)]],
                          rows2.at[1], gsem).wait()
    pltpu.make_async_copy(ex_hbm.at[pl.ds(base, B)],
                          ex2.at[pl.ds(B, B)], esem).wait()
    pltpu.make_async_copy(rows2.at[0],
                          hp_s.at[tgt_v.at[pl.ds((NB - 1) * B, B)]],
                          ssem).wait()
    plsc.subcore_barrier()

    for k in range(ROWS_PER_SUB // B):
        row0 = s * ROWS_PER_SUB + k * B
        pltpu.sync_copy(hp_s.at[pl.ds(row0, B)], rows2.at[0])
        pltpu.sync_copy(rows2.at[0], hp2_hbm.at[c, pl.ds(row0, B)])


_p5 = pl.kernel(
    _p5_body,
    out_type=[jax.ShapeDtypeStruct((NC, NPAD, DIM), jnp.float32)],
    mesh=_mesh,
    compiler_params=_sc_params,
    scratch_types=[pltpu.VMEM((2 * B,), jnp.float32),
                   pltpu.VMEM((EPW + B,), jnp.int32),
                   pltpu.VMEM((EPW,), jnp.int32),
                   pltpu.VMEM((2, B, DIM), jnp.float32),
                   pltpu.VMEM_SHARED((NPAD, DIM), jnp.float32),
                   pltpu.SemaphoreType.DMA,
                   pltpu.SemaphoreType.DMA,
                   pltpu.SemaphoreType.DMA],
)


def _p6_body(hp2_ref, d32_ref, o_ref):
    dn = jnp.sum(d32_ref[...], axis=0)
    hp = hp2_ref[0] + hp2_ref[1]
    att = jnp.where(dn > 0, 1.0 / jnp.where(dn > 0, dn, 1.0), 0.0)
    y = hp * att[:, None]
    o_ref[...] = jnp.where(y > 0, y, jnp.exp(jnp.minimum(y, 0.0)) - 1.0)


def _p6(hp2, d32):
    blk = NPAD // 8
    return pl.pallas_call(
        _p6_body,
        out_shape=jax.ShapeDtypeStruct((NPAD, DIM), jnp.float32),
        grid=(8,),
        in_specs=[
            pl.BlockSpec((NC, blk, DIM), lambda i: (0, i, 0)),
            pl.BlockSpec((NW, blk), lambda i: (0, i)),
        ],
        out_specs=pl.BlockSpec((blk, DIM), lambda i: (i, 0)),
    )(hp2, d32)


def kernel(node_features, edge_index, W, a):
    a1 = a[:DIM]
    a2 = a[DIM:]
    wa = jnp.stack([W @ a1, W @ a2], axis=1)
    w_cat = jnp.concatenate([W, wa, jnp.zeros((DIM, DIM - 2),
                                              jnp.float32)], axis=1)
    hs = _matmul(node_features, w_cat)
    h = hs[:, :DIM]
    pad = jnp.zeros((NPAD - N_NODES,), jnp.float32)
    s_pad = jnp.concatenate([hs[:, DIM], pad])
    t_pad = jnp.concatenate([hs[:, DIM + 1], pad])

    src = edge_index[0]
    tgt = edge_index[1]

    e, m32 = _p2(s_pad, t_pad, src, tgt)
    ex, d32 = _p4(m32, e, tgt)
    (hp2,) = _p5(h, ex, src, tgt)
    out = _p6(hp2, d32)
    return out[:N_NODES]


# speedup vs baseline: 1.0162x; 1.0162x over previous
import functools

import jax
import jax.numpy as jnp
from jax import lax
from jax.experimental import pallas as pl
from jax.experimental.pallas import tpu as pltpu
from jax.experimental.pallas import tpu_sc as plsc

N_NODES = 10000
N_EDGES = 320000
DIM = 128
ALPHA = 0.2

NC = 2
NS = 16
NW = NC * NS
EPW = N_EDGES // NW
NPAD = 10240
ROWS_PER_SUB = NPAD // NS
B = 80
NB = EPW // B
NEG = -3.0e38

_mesh = plsc.VectorSubcoreMesh(core_axis_name="c", subcore_axis_name="s")
_sc_params = pltpu.CompilerParams(needs_layout_passes=False)


def _mm_body(x_ref, w_ref, o_ref):
    o_ref[...] = jnp.dot(x_ref[...], w_ref[...],
                         preferred_element_type=jnp.float32)


def _matmul(x, w_cat):
    return pl.pallas_call(
        _mm_body,
        out_shape=jax.ShapeDtypeStruct((N_NODES, 2 * DIM), jnp.float32),
        grid=(10,),
        in_specs=[
            pl.BlockSpec((N_NODES // 10, DIM), lambda i: (i, 0)),
            pl.BlockSpec((DIM, 2 * DIM), lambda i: (0, 0)),
        ],
        out_specs=pl.BlockSpec((N_NODES // 10, 2 * DIM), lambda i: (i, 0)),
    )(x, w_cat)


def _seg_fold(tidx, vals, acc_ref, iota, is_sum):
    sk, sv = plsc.sort_key_val(tidx, vals)
    for d in (1, 2, 4, 8):
        lane = jnp.maximum(iota - d, 0)
        kk = sk.at[lane].get(mode="promise_in_bounds")
        vv = sv.at[lane].get(mode="promise_in_bounds")
        take = (kk == sk) & (iota >= d)
        folded = sv + vv if is_sum else jnp.maximum(sv, vv)
        sv = jnp.where(take, folded, sv)
    nxt = sk.at[jnp.minimum(iota + 1, 15)].get(mode="promise_in_bounds")
    last = (sk != nxt) | (iota == 15)
    cur = plsc.load_gather(acc_ref, [sk], mask=last)
    upd = cur + sv if is_sum else jnp.maximum(cur, sv)
    plsc.store_scatter(acc_ref, [sk], upd, mask=last)


def _p2_body(s_hbm, t_hbm, src_hbm, tgt_hbm, e_hbm, m32_hbm,
             s_v, t_v, src_v, tgt_v, e_v, m_v):
    c = lax.axis_index("c")
    s = lax.axis_index("s")
    wid = s * NC + c
    base = wid * EPW
    pltpu.sync_copy(s_hbm, s_v)
    pltpu.sync_copy(t_hbm, t_v)
    pltpu.sync_copy(src_hbm.at[pl.ds(base, EPW)], src_v)
    pltpu.sync_copy(tgt_hbm.at[pl.ds(base, EPW)], tgt_v)

    def _init(i, _):
        m_v[pl.ds(i * 16, 16)] = jnp.full((16,), NEG, jnp.float32)
        return 0
    lax.fori_loop(0, NPAD // 16, _init, 0)

    iota = lax.iota(jnp.int32, 16)

    def _step(i, _):
        off = i * 16
        sidx = src_v[pl.ds(off, 16)]
        tidx = tgt_v[pl.ds(off, 16)]
        ev = plsc.load_gather(s_v, [sidx]) + plsc.load_gather(t_v, [tidx])
        ev = jnp.where(ev > 0, ev, ALPHA * ev)
        e_v[pl.ds(off, 16)] = ev
        _seg_fold(tidx, ev, m_v, iota, is_sum=False)
        return 0
    lax.fori_loop(0, EPW // 16, _step, 0)

    pltpu.sync_copy(e_v, e_hbm.at[pl.ds(base, EPW)])
    pltpu.sync_copy(m_v, m32_hbm.at[wid])


_p2 = pl.kernel(
    _p2_body,
    out_type=[jax.ShapeDtypeStruct((N_EDGES,), jnp.float32),
              jax.ShapeDtypeStruct((NW, NPAD), jnp.float32)],
    mesh=_mesh,
    compiler_params=_sc_params,
    scratch_types=[pltpu.VMEM((NPAD,), jnp.float32),
                   pltpu.VMEM((NPAD,), jnp.float32),
                   pltpu.VMEM((EPW,), jnp.int32),
                   pltpu.VMEM((EPW,), jnp.int32),
                   pltpu.VMEM((EPW,), jnp.float32),
                   pltpu.VMEM((NPAD,), jnp.float32)],
)


SLICE = NPAD // NS


def _p4_body(m32_hbm, e_hbm, tgt_hbm, ex_hbm, d32_hbm,
             mst, mred, m_v, e_v, tgt_v, ex_v, d_v, m_s):
    c = lax.axis_index("c")
    s = lax.axis_index("s")
    wid = s * NC + c
    base = wid * EPW
    pltpu.sync_copy(m32_hbm.at[:, pl.ds(s * SLICE, SLICE)], mst)
    pltpu.sync_copy(e_hbm.at[pl.ds(base, EPW)], e_v)
    pltpu.sync_copy(tgt_hbm.at[pl.ds(base, EPW)], tgt_v)

    def _red(i, _):
        off = i * 16
        acc = mst[0, pl.ds(off, 16)]
        for w in range(1, NW):
            acc = jnp.maximum(acc, mst[w, pl.ds(off, 16)])
        mred[pl.ds(off, 16)] = acc
        return 0
    lax.fori_loop(0, SLICE // 16, _red, 0)
    pltpu.sync_copy(mred, m_s.at[pl.ds(s * SLICE, SLICE)])

    def _init(i, _):
        d_v[pl.ds(i * 16, 16)] = jnp.zeros((16,), jnp.float32)
        return 0
    lax.fori_loop(0, NPAD // 16, _init, 0)

    plsc.subcore_barrier()
    pltpu.sync_copy(m_s, m_v)

    def _step(i, _):
        off = i * 16
        tidx = tgt_v[pl.ds(off, 16)]
        ev = e_v[pl.ds(off, 16)]
        ex = jnp.exp(ev - plsc.load_gather(m_v, [tidx]))
        ex_v[pl.ds(off, 16)] = ex
        plsc.addupdate_scatter(d_v, [tidx], ex)
        return 0
    lax.fori_loop(0, EPW // 16, _step, 0)

    pltpu.sync_copy(ex_v, ex_hbm.at[pl.ds(base, EPW)])
    pltpu.sync_copy(d_v, d32_hbm.at[wid])


_p4 = pl.kernel(
    _p4_body,
    out_type=[jax.ShapeDtypeStruct((N_EDGES,), jnp.float32),
              jax.ShapeDtypeStruct((NW, NPAD), jnp.float32)],
    mesh=_mesh,
    compiler_params=_sc_params,
    scratch_types=[pltpu.VMEM((NW, SLICE), jnp.float32),
                   pltpu.VMEM((SLICE,), jnp.float32),
                   pltpu.VMEM((NPAD,), jnp.float32),
                   pltpu.VMEM((EPW,), jnp.float32),
                   pltpu.VMEM((EPW,), jnp.int32),
                   pltpu.VMEM((EPW,), jnp.float32),
                   pltpu.VMEM((NPAD,), jnp.float32),
                   pltpu.VMEM_SHARED((NPAD,), jnp.float32)],
)


def _p5_body(h_hbm, ex_hbm, src_hbm, tgt_hbm, hp2_hbm,
             ex2, src_v, tgt_v, rows2, hp_s, gsem, esem, ssem):
    c = lax.axis_index("c")
    s = lax.axis_index("s")
    wid = s * NC + c
    base = wid * EPW
    pltpu.sync_copy(src_hbm.at[pl.ds(base, EPW)], src_v.at[pl.ds(0, EPW)])
    pltpu.sync_copy(tgt_hbm.at[pl.ds(base, EPW)], tgt_v)
    zer16 = jnp.zeros((16,), jnp.int32)
    for k in range(B // 16):
        src_v[pl.ds(EPW + k * 16, 16)] = zer16

    def _zinit(i, _):
        r = i // 8
        cc = i % 8
        rows2[0, r, pl.ds(cc * 16, 16)] = jnp.zeros((16,), jnp.float32)
        rows2[1, r, pl.ds(cc * 16, 16)] = jnp.zeros((16,), jnp.float32)
        return 0
    lax.fori_loop(0, B * 8, _zinit, 0)
    for k in range(ROWS_PER_SUB // B):
        pltpu.sync_copy(rows2.at[0],
                        hp_s.at[pl.ds(s * ROWS_PER_SUB + k * B, B)])
    plsc.subcore_barrier()

    pltpu.async_copy(rows2.at[1], hp_s.at[tgt_v.at[pl.ds(0, B)]], ssem,
                     add=True)
    pltpu.async_copy(h_hbm.at[src_v.at[pl.ds(0, B)]], rows2.at[0], gsem)
    pltpu.async_copy(ex_hbm.at[pl.ds(base, B)], ex2.at[pl.ds(0, B)], esem)

    def _process(i, p):
        off = i * B
        off_n = lax.rem(off + B, EPW)
        pltpu.make_async_copy(rows2.at[1 - p],
                              hp_s.at[tgt_v.at[pl.ds(off, B)]], ssem).wait()
        pltpu.async_copy(h_hbm.at[src_v.at[pl.ds(off + B, B)]],
                         rows2.at[1 - p], gsem)
        pltpu.async_copy(ex_hbm.at[pl.ds(base + off_n, B)],
                         ex2.at[pl.ds((1 - p) * B, B)], esem)
        pltpu.make_async_copy(h_hbm.at[src_v.at[pl.ds(off, B)]],
                              rows2.at[p], gsem).wait()
        pltpu.make_async_copy(ex_hbm.at[pl.ds(base + off, B)],
                              ex2.at[pl.ds(p * B, B)], esem).wait()

        def _scale(g, _):
            ex16 = ex2[pl.ds(p * B + g * 16, 16)]
            for r16 in range(16):
                lane = jnp.full((16,), r16, jnp.int32)
                bc = ex16.at[lane].get(mode="promise_in_bounds")
                r = g * 16 + r16
                vals = [rows2[p, r, pl.ds(cc * 16, 16)] for cc in range(8)]
                for cc in range(8):
                    rows2[p, r, pl.ds(cc * 16, 16)] = vals[cc] * bc
            return 0
        lax.fori_loop(0, B // 16, _scale, 0)

        pltpu.async_copy(rows2.at[p], hp_s.at[tgt_v.at[pl.ds(off, B)]],
                        ssem, add=True)

    def _pair(j, _):
        _process(2 * j, 0)
        _process(2 * j + 1, 1)
        return 0
    lax.fori_loop(0, NB // 2, _pair, 0)
    _process(NB - 1, 0)

    pltpu.make_async_copy(h_hbm.at[src_v.at[pl.ds(NB * B, B)]],
                          rows2.at[1], gsem).wait()
    pltpu.make_async_copy(ex_hbm.at[pl.ds(base, B)],
                          ex2.at[pl.ds(B, B)], esem).wait()
    pltpu.make_async_copy(rows2.at[0],
                          hp_s.at[tgt_v.at[pl.ds((NB - 1) * B, B)]],
                          ssem).wait()
    plsc.subcore_barrier()

    for k in range(ROWS_PER_SUB // B):
        row0 = s * ROWS_PER_SUB + k * B
        pltpu.sync_copy(hp_s.at[pl.ds(row0, B)], rows2.at[0])
        pltpu.sync_copy(rows2.at[0], hp2_hbm.at[c, pl.ds(row0, B)])


_p5 = pl.kernel(
    _p5_body,
    out_type=[jax.ShapeDtypeStruct((NC, NPAD, DIM), jnp.float32)],
    mesh=_mesh,
    compiler_params=_sc_params,
    scratch_types=[pltpu.VMEM((2 * B,), jnp.float32),
                   pltpu.VMEM((EPW + B,), jnp.int32),
                   pltpu.VMEM((EPW,), jnp.int32),
                   pltpu.VMEM((2, B, DIM), jnp.float32),
                   pltpu.VMEM_SHARED((NPAD, DIM), jnp.float32),
                   pltpu.SemaphoreType.DMA,
                   pltpu.SemaphoreType.DMA,
                   pltpu.SemaphoreType.DMA],
)


def _p6_body(hp2_ref, d32_ref, o_ref):
    dn = jnp.sum(d32_ref[...], axis=0)
    hp = hp2_ref[0] + hp2_ref[1]
    att = jnp.where(dn > 0, 1.0 / jnp.where(dn > 0, dn, 1.0), 0.0)
    y = hp * att[:, None]
    o_ref[...] = jnp.where(y > 0, y, jnp.exp(jnp.minimum(y, 0.0)) - 1.0)


def _p6(hp2, d32):
    blk = NPAD // 8
    return pl.pallas_call(
        _p6_body,
        out_shape=jax.ShapeDtypeStruct((NPAD, DIM), jnp.float32),
        grid=(8,),
        in_specs=[
            pl.BlockSpec((NC, blk, DIM), lambda i: (0, i, 0)),
            pl.BlockSpec((NW, blk), lambda i: (0, i)),
        ],
        out_specs=pl.BlockSpec((blk, DIM), lambda i: (i, 0)),
    )(hp2, d32)


def kernel(node_features, edge_index, W, a):
    a1 = a[:DIM]
    a2 = a[DIM:]
    wa = jnp.stack([W @ a1, W @ a2], axis=1)
    w_cat = jnp.concatenate([W, wa, jnp.zeros((DIM, DIM - 2),
                                              jnp.float32)], axis=1)
    hs = _matmul(node_features, w_cat)
    h = hs[:, :DIM]
    pad = jnp.zeros((NPAD - N_NODES,), jnp.float32)
    s_pad = jnp.concatenate([hs[:, DIM], pad])
    t_pad = jnp.concatenate([hs[:, DIM + 1], pad])

    src = edge_index[0]
    tgt = edge_index[1]

    e, m32 = _p2(s_pad, t_pad, src, tgt)
    ex, d32 = _p4(m32, e, tgt)
    (hp2,) = _p5(h, ex, src, tgt)
    out = _p6(hp2, d32)
    return out[:N_NODES]


# speedup vs baseline: 1.0251x; 1.0088x over previous
import functools

import jax
import jax.numpy as jnp
from jax import lax
from jax.experimental import pallas as pl
from jax.experimental.pallas import tpu as pltpu
from jax.experimental.pallas import tpu_sc as plsc

N_NODES = 10000
N_EDGES = 320000
DIM = 128
ALPHA = 0.2

NC = 2
NS = 16
NW = NC * NS
EPW = N_EDGES // NW
NPAD = 10240
ROWS_PER_SUB = NPAD // NS
B = 80
NB = EPW // B
NEG = -3.0e38

_mesh = plsc.VectorSubcoreMesh(core_axis_name="c", subcore_axis_name="s")
_sc_params = pltpu.CompilerParams(needs_layout_passes=False)


def _mm_body(x_ref, w_ref, o_ref):
    o_ref[...] = jnp.dot(x_ref[...], w_ref[...],
                         preferred_element_type=jnp.float32)


def _matmul(x, w_cat):
    return pl.pallas_call(
        _mm_body,
        out_shape=jax.ShapeDtypeStruct((N_NODES, 2 * DIM), jnp.float32),
        grid=(10,),
        in_specs=[
            pl.BlockSpec((N_NODES // 10, DIM), lambda i: (i, 0)),
            pl.BlockSpec((DIM, 2 * DIM), lambda i: (0, 0)),
        ],
        out_specs=pl.BlockSpec((N_NODES // 10, 2 * DIM), lambda i: (i, 0)),
    )(x, w_cat)


def _seg_fold(tidx, vals, acc_ref, iota, is_sum):
    sk, sv = plsc.sort_key_val(tidx, vals)
    for d in (1, 2, 4, 8):
        lane = jnp.maximum(iota - d, 0)
        kk = sk.at[lane].get(mode="promise_in_bounds")
        vv = sv.at[lane].get(mode="promise_in_bounds")
        take = (kk == sk) & (iota >= d)
        folded = sv + vv if is_sum else jnp.maximum(sv, vv)
        sv = jnp.where(take, folded, sv)
    nxt = sk.at[jnp.minimum(iota + 1, 15)].get(mode="promise_in_bounds")
    last = (sk != nxt) | (iota == 15)
    cur = plsc.load_gather(acc_ref, [sk], mask=last)
    upd = cur + sv if is_sum else jnp.maximum(cur, sv)
    plsc.store_scatter(acc_ref, [sk], upd, mask=last)


def _p2_body(s_hbm, t_hbm, src_hbm, tgt_hbm, e_hbm, m32_hbm,
             s_v, t_v, src_v, tgt_v, e_v, m_v):
    c = lax.axis_index("c")
    s = lax.axis_index("s")
    wid = s * NC + c
    base = wid * EPW
    pltpu.sync_copy(s_hbm, s_v)
    pltpu.sync_copy(t_hbm, t_v)
    pltpu.sync_copy(src_hbm.at[pl.ds(base, EPW)], src_v)
    pltpu.sync_copy(tgt_hbm.at[pl.ds(base, EPW)], tgt_v)

    def _init(i, _):
        m_v[pl.ds(i * 16, 16)] = jnp.full((16,), NEG, jnp.float32)
        return 0
    lax.fori_loop(0, NPAD // 16, _init, 0)

    iota = lax.iota(jnp.int32, 16)

    def _step(i, _):
        off = i * 16
        sidx = src_v[pl.ds(off, 16)]
        tidx = tgt_v[pl.ds(off, 16)]
        ev = plsc.load_gather(s_v, [sidx]) + plsc.load_gather(t_v, [tidx])
        ev = jnp.where(ev > 0, ev, ALPHA * ev)
        e_v[pl.ds(off, 16)] = ev
        _seg_fold(tidx, ev, m_v, iota, is_sum=False)
        return 0
    lax.fori_loop(0, EPW // 16, _step, 0)

    pltpu.sync_copy(e_v, e_hbm.at[pl.ds(base, EPW)])
    pltpu.sync_copy(m_v, m32_hbm.at[wid])


_p2 = pl.kernel(
    _p2_body,
    out_type=[jax.ShapeDtypeStruct((N_EDGES,), jnp.float32),
              jax.ShapeDtypeStruct((NW, NPAD), jnp.float32)],
    mesh=_mesh,
    compiler_params=_sc_params,
    scratch_types=[pltpu.VMEM((NPAD,), jnp.float32),
                   pltpu.VMEM((NPAD,), jnp.float32),
                   pltpu.VMEM((EPW,), jnp.int32),
                   pltpu.VMEM((EPW,), jnp.int32),
                   pltpu.VMEM((EPW,), jnp.float32),
                   pltpu.VMEM((NPAD,), jnp.float32)],
)


SLICE = NPAD // NS


def _p4_body(m32_hbm, e_hbm, tgt_hbm, ex_hbm, d32_hbm,
             mst, mred, m_v, e_v, tgt_v, ex_v, d_v, m_s):
    c = lax.axis_index("c")
    s = lax.axis_index("s")
    wid = s * NC + c
    base = wid * EPW
    pltpu.sync_copy(m32_hbm.at[:, pl.ds(s * SLICE, SLICE)], mst)
    pltpu.sync_copy(e_hbm.at[pl.ds(base, EPW)], e_v)
    pltpu.sync_copy(tgt_hbm.at[pl.ds(base, EPW)], tgt_v)

    def _red(i, _):
        off = i * 16
        acc = mst[0, pl.ds(off, 16)]
        for w in range(1, NW):
            acc = jnp.maximum(acc, mst[w, pl.ds(off, 16)])
        mred[pl.ds(off, 16)] = acc
        return 0
    lax.fori_loop(0, SLICE // 16, _red, 0)
    pltpu.sync_copy(mred, m_s.at[pl.ds(s * SLICE, SLICE)])

    def _init(i, _):
        d_v[pl.ds(i * 16, 16)] = jnp.zeros((16,), jnp.float32)
        return 0
    lax.fori_loop(0, NPAD // 16, _init, 0)

    plsc.subcore_barrier()
    pltpu.sync_copy(m_s, m_v)

    def _step(i, _):
        off = i * 16
        tidx = tgt_v[pl.ds(off, 16)]
        ev = e_v[pl.ds(off, 16)]
        ex = jnp.exp(ev - plsc.load_gather(m_v, [tidx]))
        ex_v[pl.ds(off, 16)] = ex
        plsc.addupdate_scatter(d_v, [tidx], ex)
        return 0
    lax.fori_loop(0, EPW // 16, _step, 0)

    pltpu.sync_copy(ex_v, ex_hbm.at[pl.ds(base, EPW)])
    pltpu.sync_copy(d_v, d32_hbm.at[wid])


_p4 = pl.kernel(
    _p4_body,
    out_type=[jax.ShapeDtypeStruct((N_EDGES,), jnp.float32),
              jax.ShapeDtypeStruct((NW, NPAD), jnp.float32)],
    mesh=_mesh,
    compiler_params=_sc_params,
    scratch_types=[pltpu.VMEM((NW, SLICE), jnp.float32),
                   pltpu.VMEM((SLICE,), jnp.float32),
                   pltpu.VMEM((NPAD,), jnp.float32),
                   pltpu.VMEM((EPW,), jnp.float32),
                   pltpu.VMEM((EPW,), jnp.int32),
                   pltpu.VMEM((EPW,), jnp.float32),
                   pltpu.VMEM((NPAD,), jnp.float32),
                   pltpu.VMEM_SHARED((NPAD,), jnp.float32)],
)


def _p5_body(h_hbm, ex_hbm, src_hbm, tgt_hbm, hp2_hbm,
             ex2, src_v, tgt_v, rows2, hp_s, gsem, g2sem, esem, ssem):
    c = lax.axis_index("c")
    s = lax.axis_index("s")
    wid = s * NC + c
    base = wid * EPW
    pltpu.sync_copy(src_hbm.at[pl.ds(base, EPW)], src_v.at[pl.ds(0, EPW)])
    pltpu.sync_copy(tgt_hbm.at[pl.ds(base, EPW)], tgt_v)
    zer16 = jnp.zeros((16,), jnp.int32)
    for k in range(B // 16):
        src_v[pl.ds(EPW + k * 16, 16)] = zer16

    def _zinit(i, _):
        r = i // 8
        cc = i % 8
        rows2[0, r, pl.ds(cc * 16, 16)] = jnp.zeros((16,), jnp.float32)
        rows2[1, r, pl.ds(cc * 16, 16)] = jnp.zeros((16,), jnp.float32)
        return 0
    lax.fori_loop(0, B * 8, _zinit, 0)
    for k in range(ROWS_PER_SUB // B):
        pltpu.sync_copy(rows2.at[0],
                        hp_s.at[pl.ds(s * ROWS_PER_SUB + k * B, B)])
    plsc.subcore_barrier()

    pltpu.async_copy(rows2.at[1], hp_s.at[tgt_v.at[pl.ds(0, B)]], ssem,
                     add=True)
    pltpu.async_copy(h_hbm.at[src_v.at[pl.ds(0, B // 2)]],
                     rows2.at[0, pl.ds(0, B // 2)], gsem)
    pltpu.async_copy(h_hbm.at[src_v.at[pl.ds(B // 2, B // 2)]],
                     rows2.at[0, pl.ds(B // 2, B // 2)], g2sem)
    pltpu.async_copy(ex_hbm.at[pl.ds(base, B)], ex2.at[pl.ds(0, B)], esem)

    def _process(i, p):
        off = i * B
        off_n = lax.rem(off + B, EPW)
        pltpu.make_async_copy(rows2.at[1 - p],
                              hp_s.at[tgt_v.at[pl.ds(off, B)]], ssem).wait()
        pltpu.async_copy(h_hbm.at[src_v.at[pl.ds(off + B, B // 2)]],
                         rows2.at[1 - p, pl.ds(0, B // 2)], gsem)
        pltpu.async_copy(h_hbm.at[src_v.at[pl.ds(off + B + B // 2, B // 2)]],
                         rows2.at[1 - p, pl.ds(B // 2, B // 2)], g2sem)
        pltpu.async_copy(ex_hbm.at[pl.ds(base + off_n, B)],
                         ex2.at[pl.ds((1 - p) * B, B)], esem)
        pltpu.make_async_copy(h_hbm.at[src_v.at[pl.ds(off, B // 2)]],
                              rows2.at[p, pl.ds(0, B // 2)], gsem).wait()
        pltpu.make_async_copy(h_hbm.at[src_v.at[pl.ds(off + B // 2, B // 2)]],
                              rows2.at[p, pl.ds(B // 2, B // 2)],
                              g2sem).wait()
        pltpu.make_async_copy(ex_hbm.at[pl.ds(base + off, B)],
                              ex2.at[pl.ds(p * B, B)], esem).wait()

        def _scale(g, _):
            ex16 = ex2[pl.ds(p * B + g * 16, 16)]
            for r16 in range(16):
                lane = jnp.full((16,), r16, jnp.int32)
                bc = ex16.at[lane].get(mode="promise_in_bounds")
                r = g * 16 + r16
                vals = [rows2[p, r, pl.ds(cc * 16, 16)] for cc in range(8)]
                for cc in range(8):
                    rows2[p, r, pl.ds(cc * 16, 16)] = vals[cc] * bc
            return 0
        lax.fori_loop(0, B // 16, _scale, 0)

        pltpu.async_copy(rows2.at[p], hp_s.at[tgt_v.at[pl.ds(off, B)]],
                        ssem, add=True)

    def _pair(j, _):
        _process(2 * j, 0)
        _process(2 * j + 1, 1)
        return 0
    lax.fori_loop(0, NB // 2, _pair, 0)
    _process(NB - 1, 0)

    pltpu.make_async_copy(h_hbm.at[src_v.at[pl.ds(NB * B, B // 2)]],
                          rows2.at[1, pl.ds(0, B // 2)], gsem).wait()
    pltpu.make_async_copy(h_hbm.at[src_v.at[pl.ds(NB * B + B // 2, B // 2)]],
                          rows2.at[1, pl.ds(B // 2, B // 2)], g2sem).wait()
    pltpu.make_async_copy(ex_hbm.at[pl.ds(base, B)],
                          ex2.at[pl.ds(B, B)], esem).wait()
    pltpu.make_async_copy(rows2.at[0],
                          hp_s.at[tgt_v.at[pl.ds((NB - 1) * B, B)]],
                          ssem).wait()
    plsc.subcore_barrier()

    for k in range(ROWS_PER_SUB // B):
        row0 = s * ROWS_PER_SUB + k * B
        pltpu.sync_copy(hp_s.at[pl.ds(row0, B)], rows2.at[0])
        pltpu.sync_copy(rows2.at[0], hp2_hbm.at[c, pl.ds(row0, B)])


_p5 = pl.kernel(
    _p5_body,
    out_type=[jax.ShapeDtypeStruct((NC, NPAD, DIM), jnp.float32)],
    mesh=_mesh,
    compiler_params=_sc_params,
    scratch_types=[pltpu.VMEM((2 * B,), jnp.float32),
                   pltpu.VMEM((EPW + B,), jnp.int32),
                   pltpu.VMEM((EPW,), jnp.int32),
                   pltpu.VMEM((2, B, DIM), jnp.float32),
                   pltpu.VMEM_SHARED((NPAD, DIM), jnp.float32),
                   pltpu.SemaphoreType.DMA,
                   pltpu.SemaphoreType.DMA,
                   pltpu.SemaphoreType.DMA,
                   pltpu.SemaphoreType.DMA],
)


def _p6_body(hp2_ref, d32_ref, o_ref):
    dn = jnp.sum(d32_ref[...], axis=0)
    hp = hp2_ref[0] + hp2_ref[1]
    att = jnp.where(dn > 0, 1.0 / jnp.where(dn > 0, dn, 1.0), 0.0)
    y = hp * att[:, None]
    o_ref[...] = jnp.where(y > 0, y, jnp.exp(jnp.minimum(y, 0.0)) - 1.0)


def _p6(hp2, d32):
    blk = NPAD // 8
    return pl.pallas_call(
        _p6_body,
        out_shape=jax.ShapeDtypeStruct((NPAD, DIM), jnp.float32),
        grid=(8,),
        in_specs=[
            pl.BlockSpec((NC, blk, DIM), lambda i: (0, i, 0)),
            pl.BlockSpec((NW, blk), lambda i: (0, i)),
        ],
        out_specs=pl.BlockSpec((blk, DIM), lambda i: (i, 0)),
    )(hp2, d32)


def kernel(node_features, edge_index, W, a):
    a1 = a[:DIM]
    a2 = a[DIM:]
    wa = jnp.stack([W @ a1, W @ a2], axis=1)
    w_cat = jnp.concatenate([W, wa, jnp.zeros((DIM, DIM - 2),
                                              jnp.float32)], axis=1)
    hs = _matmul(node_features, w_cat)
    h = hs[:, :DIM]
    pad = jnp.zeros((NPAD - N_NODES,), jnp.float32)
    s_pad = jnp.concatenate([hs[:, DIM], pad])
    t_pad = jnp.concatenate([hs[:, DIM + 1], pad])

    src = edge_index[0]
    tgt = edge_index[1]

    e, m32 = _p2(s_pad, t_pad, src, tgt)
    ex, d32 = _p4(m32, e, tgt)
    (hp2,) = _p5(h, ex, src, tgt)
    out = _p6(hp2, d32)
    return out[:N_NODES]
